# Initial kernel scaffold; baseline (speedup 1.0000x reference)
import jax
import jax.numpy as jnp
from jax.experimental import pallas as pl


def kernel(table, ln1_g, ln1_b, W1, b1, ln2_g, ln2_b, W2, b2):
    raise NotImplementedError("write your pallas kernel here")


# speedup vs baseline: 59.6029x; 59.6029x over previous
import functools

import jax
import jax.numpy as jnp
from jax import lax
from jax.experimental import pallas as pl
from jax.experimental.pallas import tpu as pltpu
from jax.experimental.pallas import tpu_sc as plsc

B = 16
ROWS = 512
D = 26 * 64
H = 16
CAP = 64
REG = 0.01
EPS = 1e-5


def _score_body(t_ref, g1_ref, bb1_ref, w1_ref, b1_ref, g2_ref, bb2_ref,
                w2_ref, b2_ref, out_ref):
    t = t_ref[0]
    m = jnp.mean(t, axis=-1, keepdims=True)
    v = jnp.mean((t - m) ** 2, axis=-1, keepdims=True)
    h = (t - m) * lax.rsqrt(v + EPS) * g1_ref[0] + bb1_ref[0]
    h = lax.dot_general(h, w1_ref[...], (((1,), (1,)), ((), ())),
                        preferred_element_type=jnp.float32)
    h = jnp.maximum(h + b1_ref[0], 0.0)
    m2 = jnp.mean(h, axis=-1, keepdims=True)
    v2 = jnp.mean((h - m2) ** 2, axis=-1, keepdims=True)
    h = (h - m2) * lax.rsqrt(v2 + EPS) * g2_ref[0] + bb2_ref[0]
    s = jnp.sum(h * w2_ref[0], axis=-1) + b2_ref[0, 0]
    out_ref[0, 0, :] = s


def _scores(t, g1, bb1, W1, b1, g2, bb2, w2, b2):
    full = lambda shape: pl.BlockSpec(shape, lambda i: (0,) * len(shape))
    return pl.pallas_call(
        _score_body,
        grid=(B,),
        in_specs=[
            pl.BlockSpec((1, ROWS, D), lambda i: (i, 0, 0)),
            full((1, D)), full((1, D)), full((H, D)), full((1, H)),
            full((1, H)), full((1, H)), full((1, H)), full((1, 1)),
        ],
        out_specs=pl.BlockSpec((1, 1, ROWS), lambda i: (i, 0, 0)),
        out_shape=jax.ShapeDtypeStruct((B, 1, ROWS), jnp.float32),
    )(t, g1, bb1, W1, b1, g2, bb2, w2, b2).reshape(B, ROWS)


def _sort_body(sc_ref, s_out, p_out):
    sc = sc_ref[...]
    mn = jnp.min(sc, axis=1, keepdims=True)
    mx = jnp.max(sc, axis=1, keepdims=True)
    keys = (sc - mn) / (mx - mn) * (100.0 / REG)
    vals = lax.broadcasted_iota(jnp.int32, (B, ROWS), 1)
    idx = vals
    k = 2
    while k <= ROWS:
        j = k // 2
        while j >= 1:
            lower = (idx & j) == 0
            pk = jnp.where(lower, jnp.roll(keys, -j, axis=1),
                           jnp.roll(keys, j, axis=1))
            pv = jnp.where(lower, jnp.roll(vals, -j, axis=1),
                           jnp.roll(vals, j, axis=1))
            dirb = (idx & k) != 0
            ws = (dirb & lower) | (~dirb & ~lower)
            le = keys <= pk
            lt = keys < pk
            cond_small = (lower & le) | (~lower & lt)
            cond_big = (lower & ~le) | (~lower & ~lt)
            keep = (ws & cond_small) | (~ws & cond_big)
            keys = jnp.where(keep, keys, pk)
            vals = jnp.where(keep, vals, pv)
            j //= 2
        k *= 2
    s_out[...] = keys
    p_out[...] = vals


def _sort(scores):
    return pl.pallas_call(
        _sort_body,
        out_shape=(jax.ShapeDtypeStruct((B, ROWS), jnp.float32),
                   jax.ShapeDtypeStruct((B, ROWS), jnp.int32)),
    )(scores)


def _sc_body(s_hbm, p_hbm, ranks_hbm, ridx_hbm,
             sv, ssum, slen, outt, permv, perm15, outrow, ridxrow):
    cid = lax.axis_index("c")
    sid = lax.axis_index("s")
    wid = sid * 2 + cid

    @pl.when(wid < B)
    def _():
        pltpu.sync_copy(s_hbm, sv)
        lanes = lax.iota(jnp.int32, 16)
        ones = jnp.ones((16,), jnp.float32)

        def pav_step(i, sp):
            ii = jnp.full((16,), i, jnp.int32)
            y_i = plsc.load_gather(sv, [lanes * ROWS + ii]) - (
                jnp.float32(ROWS) - i.astype(jnp.float32))
            plsc.store_scatter(ssum, [sp * 16 + lanes], y_i)
            plsc.store_scatter(slen, [sp * 16 + lanes], ones)
            sp = sp + 1

            def mbody(c):
                spc, _ = c
                i1 = jnp.maximum(spc - 1, 0) * 16 + lanes
                i2 = jnp.maximum(spc - 2, 0) * 16 + lanes
                ts = plsc.load_gather(ssum, [i1])
                tl = plsc.load_gather(slen, [i1])
                ps = plsc.load_gather(ssum, [i2])
                pls = plsc.load_gather(slen, [i2])
                need = (spc >= 2) & (ts * pls > ps * tl)
                plsc.store_scatter(ssum, [i2], ps + ts, mask=need)
                plsc.store_scatter(slen, [i2], pls + tl, mask=need)
                return spc - need.astype(jnp.int32), jnp.any(need)

            sp, _ = lax.while_loop(lambda c: c[1], mbody,
                                   (sp, jnp.bool_(True)))
            return sp

        lax.fori_loop(0, ROWS, pav_step, jnp.zeros((16,), jnp.int32))

        def exp_step(i, c):
            cp, endp, avg = c
            ii = jnp.full((16,), i, jnp.int32)
            adv = i.astype(jnp.float32) >= endp
            ns = plsc.load_gather(ssum, [cp * 16 + lanes])
            nl = plsc.load_gather(slen, [cp * 16 + lanes])
            avg = jnp.where(adv, ns / nl, avg)
            endp = jnp.where(adv, endp + nl, endp)
            cp = cp + adv.astype(jnp.int32)
            y_i = plsc.load_gather(sv, [lanes * ROWS + ii]) - (
                jnp.float32(ROWS) - i.astype(jnp.float32))
            out = y_i - avg + (jnp.float32(ROWS) - i.astype(jnp.float32))
            plsc.store_scatter(outt, [ii * 16 + lanes], out)
            return cp, endp, avg

        lax.fori_loop(0, ROWS, exp_step,
                      (jnp.zeros((16,), jnp.int32),
                       jnp.zeros((16,), jnp.float32),
                       jnp.zeros((16,), jnp.float32)))

        pltpu.sync_copy(p_hbm.at[wid], permv)
        pltpu.sync_copy(p_hbm.at[B - 1], perm15)
        bvec = jnp.full((16,), wid, jnp.int32)
        for c in range(ROWS // 16):
            rows = c * 16 + lanes
            vals = plsc.load_gather(outt, [rows * 16 + bvec])
            pos = permv[pl.ds(c * 16, 16)]
            plsc.store_scatter(outrow, [pos], vals)
            rvals = ((jnp.int32(ROWS - 1) - rows) >> 6) + 1
            p15 = perm15[pl.ds(c * 16, 16)]
            plsc.store_scatter(ridxrow, [p15], rvals)
        pltpu.sync_copy(outrow, ranks_hbm.at[wid])
        pltpu.sync_copy(ridxrow, ridx_hbm.at[wid])


def _sc_stage(s_desc, perm):
    mesh = plsc.VectorSubcoreMesh(core_axis_name="c", subcore_axis_name="s")
    kfn = pl.kernel(
        _sc_body,
        mesh=mesh,
        compiler_params=pltpu.CompilerParams(needs_layout_passes=False),
        out_type=(jax.ShapeDtypeStruct((B, ROWS), jnp.float32),
                  jax.ShapeDtypeStruct((B, ROWS), jnp.int32)),
        scratch_types=[
            pltpu.VMEM((B * ROWS,), jnp.float32),
            pltpu.VMEM((ROWS * 16,), jnp.float32),
            pltpu.VMEM((ROWS * 16,), jnp.float32),
            pltpu.VMEM((ROWS * 16,), jnp.float32),
            pltpu.VMEM((ROWS,), jnp.int32),
            pltpu.VMEM((ROWS,), jnp.int32),
            pltpu.VMEM((ROWS,), jnp.float32),
            pltpu.VMEM((ROWS,), jnp.int32),
        ],
    )
    return kfn(s_desc.reshape(B * ROWS), perm)


def kernel(table, ln1_g, ln1_b, W1, b1, ln2_g, ln2_b, W2, b2):
    t = table.reshape(B, ROWS, D)
    scores = _scores(t, ln1_g.reshape(1, D), ln1_b.reshape(1, D), W1,
                     b1.reshape(1, H), ln2_g.reshape(1, H),
                     ln2_b.reshape(1, H), W2.reshape(1, H), b2.reshape(1, 1))
    s_desc, perm = _sort(scores)
    ranks, ridx = _sc_stage(s_desc, perm)
    return ranks.reshape(B, ROWS, 1), ridx.reshape(B, ROWS, 1)


# speedup vs baseline: 64.0799x; 1.0751x over previous
import functools

import jax
import jax.numpy as jnp
from jax import lax
from jax.experimental import pallas as pl
from jax.experimental.pallas import tpu as pltpu
from jax.experimental.pallas import tpu_sc as plsc

B = 16
ROWS = 512
D = 26 * 64
H = 16
CAP = 64
REG = 0.01
EPS = 1e-5


def _score_body(t_ref, g1_ref, bb1_ref, w1_ref, b1_ref, g2_ref, bb2_ref,
                w2_ref, b2_ref, out_ref):
    t = t_ref[0]
    m = jnp.mean(t, axis=-1, keepdims=True)
    v = jnp.mean((t - m) ** 2, axis=-1, keepdims=True)
    h = (t - m) * lax.rsqrt(v + EPS) * g1_ref[0] + bb1_ref[0]
    h = lax.dot_general(h, w1_ref[...], (((1,), (1,)), ((), ())),
                        preferred_element_type=jnp.float32)
    h = jnp.maximum(h + b1_ref[0], 0.0)
    m2 = jnp.mean(h, axis=-1, keepdims=True)
    v2 = jnp.mean((h - m2) ** 2, axis=-1, keepdims=True)
    h = (h - m2) * lax.rsqrt(v2 + EPS) * g2_ref[0] + bb2_ref[0]
    s = jnp.sum(h * w2_ref[0], axis=-1) + b2_ref[0, 0]
    out_ref[0, 0, :] = s


def _scores(t, g1, bb1, W1, b1, g2, bb2, w2, b2):
    full = lambda shape: pl.BlockSpec(shape, lambda i: (0,) * len(shape))
    return pl.pallas_call(
        _score_body,
        grid=(B,),
        in_specs=[
            pl.BlockSpec((1, ROWS, D), lambda i: (i, 0, 0)),
            full((1, D)), full((1, D)), full((H, D)), full((1, H)),
            full((1, H)), full((1, H)), full((1, H)), full((1, 1)),
        ],
        out_specs=pl.BlockSpec((1, 1, ROWS), lambda i: (i, 0, 0)),
        out_shape=jax.ShapeDtypeStruct((B, 1, ROWS), jnp.float32),
    )(t, g1, bb1, W1, b1, g2, bb2, w2, b2).reshape(B, ROWS)


def _sort_body(sc_ref, s_out, p_out):
    sc = sc_ref[...]
    mn = jnp.min(sc, axis=1, keepdims=True)
    mx = jnp.max(sc, axis=1, keepdims=True)
    keys = (sc - mn) / (mx - mn) * (100.0 / REG)
    vals = lax.broadcasted_iota(jnp.int32, (B, ROWS), 1)
    idx = vals
    k = 2
    while k <= ROWS:
        j = k // 2
        while j >= 1:
            lower = (idx & j) == 0
            pk = jnp.where(lower, jnp.roll(keys, -j, axis=1),
                           jnp.roll(keys, j, axis=1))
            pv = jnp.where(lower, jnp.roll(vals, -j, axis=1),
                           jnp.roll(vals, j, axis=1))
            dirb = (idx & k) != 0
            ws = (dirb & lower) | (~dirb & ~lower)
            le = keys <= pk
            lt = keys < pk
            cond_small = (lower & le) | (~lower & lt)
            cond_big = (lower & ~le) | (~lower & ~lt)
            keep = (ws & cond_small) | (~ws & cond_big)
            keys = jnp.where(keep, keys, pk)
            vals = jnp.where(keep, vals, pv)
            j //= 2
        k *= 2
    s_out[...] = keys.T
    p_out[...] = vals


def _sort(scores):
    return pl.pallas_call(
        _sort_body,
        out_shape=(jax.ShapeDtypeStruct((ROWS, B), jnp.float32),
                   jax.ShapeDtypeStruct((B, ROWS), jnp.int32)),
    )(scores)


def _sc_body(s_hbm, p_hbm, ranks_hbm, ridx_hbm,
             sv, ssum, slen, outt, permv, perm15, outrow, ridxrow):
    cid = lax.axis_index("c")
    sid = lax.axis_index("s")
    wid = sid * 2 + cid

    @pl.when(wid < B)
    def _():
        pltpu.sync_copy(s_hbm, sv)
        lanes = lax.iota(jnp.int32, 16)

        def pav_step(i, c):
            sp, ts, tl = c
            y_i = sv[pl.ds(i * 16, 16)] - (
                jnp.float32(ROWS) - i.astype(jnp.float32))
            plsc.store_scatter(ssum, [sp * 16 + lanes], ts)
            plsc.store_scatter(slen, [sp * 16 + lanes], tl)
            sp = sp + 1
            ts = y_i
            tl = jnp.ones((16,), jnp.float32)

            def mbody(mc):
                spc, mts, mtl, _ = mc
                i1 = jnp.maximum(spc - 1, 0) * 16 + lanes
                ps = plsc.load_gather(ssum, [i1])
                pls = plsc.load_gather(slen, [i1])
                need = (spc >= 1) & (mts * pls > ps * mtl)
                mts = jnp.where(need, mts + ps, mts)
                mtl = jnp.where(need, mtl + pls, mtl)
                return (spc - need.astype(jnp.int32), mts, mtl,
                        jnp.any(need))

            sp, ts, tl, _ = lax.while_loop(
                lambda mc: mc[3], mbody, (sp, ts, tl, jnp.bool_(True)))
            return sp, ts, tl

        sp0 = jnp.zeros((16,), jnp.int32)
        ts0 = sv[pl.ds(0, 16)] - jnp.float32(ROWS)
        tl0 = jnp.ones((16,), jnp.float32)
        sp, ts, tl = lax.fori_loop(1, ROWS, pav_step, (sp0, ts0, tl0))
        plsc.store_scatter(ssum, [sp * 16 + lanes], ts)
        plsc.store_scatter(slen, [sp * 16 + lanes], tl)

        def exp_step(i, c):
            cp, endp, avg = c
            adv = i.astype(jnp.float32) >= endp
            ns = plsc.load_gather(ssum, [cp * 16 + lanes])
            nl = plsc.load_gather(slen, [cp * 16 + lanes])
            avg = jnp.where(adv, ns / nl, avg)
            endp = jnp.where(adv, endp + nl, endp)
            cp = cp + adv.astype(jnp.int32)
            outt[pl.ds(i * 16, 16)] = sv[pl.ds(i * 16, 16)] - avg
            return cp, endp, avg

        lax.fori_loop(0, ROWS, exp_step,
                      (jnp.zeros((16,), jnp.int32),
                       jnp.zeros((16,), jnp.float32),
                       jnp.zeros((16,), jnp.float32)))

        pltpu.sync_copy(p_hbm.at[wid], permv)
        pltpu.sync_copy(p_hbm.at[B - 1], perm15)
        bvec = jnp.full((16,), wid, jnp.int32)
        for c in range(ROWS // 16):
            rows = c * 16 + lanes
            vals = plsc.load_gather(outt, [rows * 16 + bvec])
            pos = permv[pl.ds(c * 16, 16)]
            plsc.store_scatter(outrow, [pos], vals)
            rvals = ((jnp.int32(ROWS - 1) - rows) >> 6) + 1
            p15 = perm15[pl.ds(c * 16, 16)]
            plsc.store_scatter(ridxrow, [p15], rvals)
        pltpu.sync_copy(outrow, ranks_hbm.at[wid])
        pltpu.sync_copy(ridxrow, ridx_hbm.at[wid])


def _sc_stage(s_desc, perm):
    mesh = plsc.VectorSubcoreMesh(core_axis_name="c", subcore_axis_name="s")
    kfn = pl.kernel(
        _sc_body,
        mesh=mesh,
        compiler_params=pltpu.CompilerParams(needs_layout_passes=False),
        out_type=(jax.ShapeDtypeStruct((B, ROWS), jnp.float32),
                  jax.ShapeDtypeStruct((B, ROWS), jnp.int32)),
        scratch_types=[
            pltpu.VMEM((B * ROWS,), jnp.float32),
            pltpu.VMEM((ROWS * 16,), jnp.float32),
            pltpu.VMEM((ROWS * 16,), jnp.float32),
            pltpu.VMEM((ROWS * 16,), jnp.float32),
            pltpu.VMEM((ROWS,), jnp.int32),
            pltpu.VMEM((ROWS,), jnp.int32),
            pltpu.VMEM((ROWS,), jnp.float32),
            pltpu.VMEM((ROWS,), jnp.int32),
        ],
    )
    return kfn(s_desc.reshape(B * ROWS), perm)


def kernel(table, ln1_g, ln1_b, W1, b1, ln2_g, ln2_b, W2, b2):
    t = table.reshape(B, ROWS, D)
    scores = _scores(t, ln1_g.reshape(1, D), ln1_b.reshape(1, D), W1,
                     b1.reshape(1, H), ln2_g.reshape(1, H),
                     ln2_b.reshape(1, H), W2.reshape(1, H), b2.reshape(1, 1))
    s_desc, perm = _sort(scores)
    ranks, ridx = _sc_stage(s_desc, perm)
    return ranks.reshape(B, ROWS, 1), ridx.reshape(B, ROWS, 1)
